# Initial kernel scaffold; baseline (speedup 1.0000x reference)
import jax
import jax.numpy as jnp
from jax.experimental import pallas as pl


def kernel(input, weight):
    raise NotImplementedError("write your pallas kernel here")


# speedup vs baseline: 15.2042x; 15.2042x over previous
import functools

import jax
import jax.numpy as jnp
from jax import lax
from jax.experimental import pallas as pl
from jax.experimental.pallas import tpu as pltpu
from jax.experimental.pallas import tpu_sc as plsc

B = 4
NUM_EMB = 100000
D = 32
SEQ = 4096 * 50
TOTAL = B * SEQ

NC = 2
NS = 16
NW = NC * NS
PER_W = TOTAL // NW
WPB = NW // B

STREAM = 128
C = 1024
K = C // STREAM
NCHUNK = PER_W // C


def _emb_body(idx_hbm, tab_hbm, out_hbm, idx_v, rows_v, sem):
    wid = lax.axis_index("s") * NC + lax.axis_index("c")
    b = wid // WPB
    off = (b * NUM_EMB).astype(jnp.int32)
    offv = jnp.full((16,), off, jnp.int32)
    base = wid * PER_W

    def chunk(g, carry):
        cbase = pl.multiple_of(base + g * C, STREAM * 8)
        rbase = pl.multiple_of((base + g * C) // STREAM, 8)
        pltpu.sync_copy(idx_hbm.at[pl.ds(rbase, K)], idx_v)
        for k in range(K):
            for i in range(STREAM // 16):
                sl = (k, pl.ds(i * 16, 16))
                idx_v[sl] = idx_v[sl] + offv
        cps = [
            pltpu.async_copy(
                tab_hbm.at[idx_v.at[k]],
                rows_v.at[pl.ds(k * STREAM, STREAM)],
                sem,
            )
            for k in range(K)
        ]
        for cp in cps:
            cp.wait()
        pltpu.sync_copy(rows_v, out_hbm.at[pl.ds(cbase, C)])
        return carry

    lax.fori_loop(0, NCHUNK, chunk, 0)


@functools.partial(jax.jit, static_argnames=())
def _run(idx2, table):
    mesh = plsc.VectorSubcoreMesh(core_axis_name="c", subcore_axis_name="s")
    f = functools.partial(
        pl.kernel,
        mesh=mesh,
        out_type=jax.ShapeDtypeStruct((TOTAL, D), jnp.float32),
        scratch_types=[
            pltpu.VMEM((K, STREAM), jnp.int32),
            pltpu.VMEM((C, D), jnp.float32),
            pltpu.SemaphoreType.DMA,
        ],
        compiler_params=pltpu.CompilerParams(use_tc_tiling_on_sc=False),
    )(_emb_body)
    return f(idx2, table)


def kernel(input, weight):
    idx2 = input.reshape(TOTAL // STREAM, STREAM).astype(jnp.int32)
    table = weight.reshape(B * NUM_EMB, D)
    out = _run(idx2, table)
    return out.reshape(B, 4096, 50, D)


# speedup vs baseline: 15.5049x; 1.0198x over previous
import functools

import jax
import jax.numpy as jnp
from jax import lax
from jax.experimental import pallas as pl
from jax.experimental.pallas import tpu as pltpu
from jax.experimental.pallas import tpu_sc as plsc

B = 4
NUM_EMB = 100000
D = 32
SEQ = 4096 * 50
TOTAL = B * SEQ

NC = 2
NS = 16
NW = NC * NS
PER_W = TOTAL // NW
WPB = NW // B

STREAM = 128
C = 1024
K = C // STREAM
NCHUNK = PER_W // C
NPAIR = NCHUNK // 2


def _emb_body(idx_hbm, tab_hbm, out_hbm, idx_v, rows_v,
              isem0, isem1, gsem, osem0, osem1):
    wid = lax.axis_index("s") * NC + lax.axis_index("c")
    b = wid // WPB
    off = (b * NUM_EMB).astype(jnp.int32)
    offv = jnp.full((16,), off, jnp.int32)
    base = wid * PER_W
    isems = (isem0, isem1)
    osems = (osem0, osem1)

    def start_idx_load(g, slot):
        rbase = pl.multiple_of((base + g * C) // STREAM, 8)
        pltpu.async_copy(idx_hbm.at[pl.ds(rbase, K)], idx_v.at[slot],
                         isems[slot])

    def wait_idx_load(slot):
        pltpu.make_async_copy(
            idx_hbm.at[pl.ds(0, K)], idx_v.at[slot], isems[slot]).wait()

    def wait_store(slot):
        pltpu.make_async_copy(
            rows_v.at[slot], out_hbm.at[pl.ds(0, C)], osems[slot]).wait()

    def do_chunk(g, slot, first, last):
        if not last:
            start_idx_load(g + 1, 1 - slot)
        wait_idx_load(slot)
        for k in range(K):
            for i in range(STREAM // 16):
                sl = pl.ds(i * 16, 16)
                idx_v[slot, k, sl] = idx_v[slot, k, sl] + offv
        if not first:
            wait_store(slot)
        cps = [
            pltpu.async_copy(
                tab_hbm.at[idx_v.at[slot, k]],
                rows_v.at[slot].at[pl.ds(k * STREAM, STREAM)],
                gsem,
            )
            for k in range(K)
        ]
        for cp in cps:
            cp.wait()
        cbase = pl.multiple_of(base + g * C, C)
        pltpu.async_copy(rows_v.at[slot], out_hbm.at[pl.ds(cbase, C)],
                         osems[slot])

    start_idx_load(0, 0)

    def pair(h, carry):
        g0 = h * 2
        do_chunk(g0, 0, False, False)
        do_chunk(g0 + 1, 1, False, False)
        return carry

    do_chunk(0, 0, True, False)
    do_chunk(1, 1, True, False)
    lax.fori_loop(1, NPAIR, pair, 0)
    do_chunk(NCHUNK - 1, 0, False, True)
    wait_store(1)
    wait_store(0)


@jax.jit
def _run(idx2, table):
    mesh = plsc.VectorSubcoreMesh(core_axis_name="c", subcore_axis_name="s")
    f = functools.partial(
        pl.kernel,
        mesh=mesh,
        out_type=jax.ShapeDtypeStruct((TOTAL, D), jnp.float32),
        scratch_types=[
            pltpu.VMEM((2, K, STREAM), jnp.int32),
            pltpu.VMEM((2, C, D), jnp.float32),
            pltpu.SemaphoreType.DMA,
            pltpu.SemaphoreType.DMA,
            pltpu.SemaphoreType.DMA,
            pltpu.SemaphoreType.DMA,
            pltpu.SemaphoreType.DMA,
        ],
        compiler_params=pltpu.CompilerParams(use_tc_tiling_on_sc=False),
    )(_emb_body)
    return f(idx2, table)


def kernel(input, weight):
    idx2 = input.reshape(TOTAL // STREAM, STREAM).astype(jnp.int32)
    table = weight.reshape(B * NUM_EMB, D)
    out = _run(idx2, table)
    return out.reshape(B, 4096, 50, D)


# speedup vs baseline: 29.1601x; 1.8807x over previous
import functools

import jax
import jax.numpy as jnp
from jax import lax
from jax.experimental import pallas as pl
from jax.experimental.pallas import tpu as pltpu
from jax.experimental.pallas import tpu_sc as plsc

B = 4
NUM_EMB = 100000
D = 32
R = 4096
CC = 50

NC = 2
NS = 16
NW = NC * NS
WPB = NW // B
ROWS_W = R // WPB

RC = 32
NCHUNK = ROWS_W // RC


def _emb_body(idx_hbm, tab_hbm, out_hbm, idx_v, rows_v,
              isem0, isem1, gsem, osem0, osem1):
    wid = lax.axis_index("s") * NC + lax.axis_index("c")
    b = wid // WPB
    j = wid % WPB
    r_base = j * ROWS_W
    isems = (isem0, isem1)
    osems = (osem0, osem1)

    def start_idx_load(g, slot):
        r0 = pl.multiple_of(r_base + g * RC, 8)
        pltpu.async_copy(idx_hbm.at[b, pl.ds(r0, RC)], idx_v.at[slot],
                         isems[slot])

    def wait_idx_load(slot):
        pltpu.make_async_copy(
            idx_hbm.at[0, pl.ds(0, RC)], idx_v.at[slot], isems[slot]).wait()

    def wait_store(slot):
        pltpu.make_async_copy(
            rows_v.at[slot], out_hbm.at[0, pl.ds(0, RC)], osems[slot]).wait()

    def do_chunk(g, slot, first, last):
        if not last:
            @pl.when(g + 1 < NCHUNK)
            def _():
                start_idx_load(g + 1, 1 - slot)
        wait_idx_load(slot)
        if not first:
            wait_store(slot)
        cps = [
            pltpu.async_copy(
                tab_hbm.at[b].at[idx_v.at[slot, r]],
                rows_v.at[slot, r],
                gsem,
            )
            for r in range(RC)
        ]
        for cp in cps:
            cp.wait()
        r0 = pl.multiple_of(r_base + g * RC, 8)
        pltpu.async_copy(rows_v.at[slot], out_hbm.at[b, pl.ds(r0, RC)],
                         osems[slot])

    start_idx_load(0, 0)
    do_chunk(0, 0, True, False)
    do_chunk(1, 1, True, False)

    def pair(h, carry):
        g0 = h * 2
        do_chunk(g0, 0, False, False)
        do_chunk(g0 + 1, 1, False, False)
        return carry

    lax.fori_loop(1, NCHUNK // 2, pair, 0)
    wait_store(0)
    wait_store(1)


@jax.jit
def _run(idx, table):
    mesh = plsc.VectorSubcoreMesh(core_axis_name="c", subcore_axis_name="s")
    f = functools.partial(
        pl.kernel,
        mesh=mesh,
        out_type=jax.ShapeDtypeStruct((B, R, CC, D), jnp.float32),
        scratch_types=[
            pltpu.VMEM((2, RC, CC), jnp.int32),
            pltpu.VMEM((2, RC, CC, D), jnp.float32),
            pltpu.SemaphoreType.DMA,
            pltpu.SemaphoreType.DMA,
            pltpu.SemaphoreType.DMA,
            pltpu.SemaphoreType.DMA,
            pltpu.SemaphoreType.DMA,
        ],
        compiler_params=pltpu.CompilerParams(use_tc_tiling_on_sc=False),
    )(_emb_body)
    return f(idx, table)


def kernel(input, weight):
    return _run(input, weight)


# speedup vs baseline: 36.2082x; 1.2417x over previous
import functools

import jax
import jax.numpy as jnp
from jax import lax
from jax.experimental import pallas as pl
from jax.experimental.pallas import tpu as pltpu
from jax.experimental.pallas import tpu_sc as plsc

B = 4
NUM_EMB = 100000
D = 32
R = 4096
CC = 50

NC = 2
NS = 16
NW = NC * NS
WPB = NW // B
ROWS_W = R // WPB

RC = 32
NCHUNK = ROWS_W // RC


def _emb_body(idx_hbm, tab_hbm, out_hbm, idx_v, rows_v,
              isem0, isem1, gsem, osem0, osem1):
    wid = lax.axis_index("s") * NC + lax.axis_index("c")
    b = wid // WPB
    j = wid % WPB
    r_base = j * ROWS_W
    isems = (isem0, isem1)
    osems = (osem0, osem1)

    def start_idx_load(g, slot):
        r0 = pl.multiple_of(r_base + g * RC, 8)
        pltpu.async_copy(idx_hbm.at[b, pl.ds(r0, RC)], idx_v.at[slot],
                         isems[slot])

    def wait_idx_load(slot):
        pltpu.make_async_copy(
            idx_hbm.at[0, pl.ds(0, RC)], idx_v.at[slot], isems[slot]).wait()

    def wait_store(slot):
        pltpu.make_async_copy(
            rows_v.at[slot], out_hbm.at[0, pl.ds(0, RC)], osems[slot]).wait()

    def do_chunk(g, slot, first, last):
        if not last:
            @pl.when(g + 1 < NCHUNK)
            def _():
                start_idx_load(g + 1, 1 - slot)
        wait_idx_load(slot)
        if not first:
            wait_store(slot)
        cps = [
            pltpu.async_copy(
                tab_hbm.at[b].at[idx_v.at[slot, r]],
                rows_v.at[slot, r],
                gsem,
            )
            for r in range(RC)
        ]
        for cp in cps:
            cp.wait()
        r0 = pl.multiple_of(r_base + g * RC, 8)
        pltpu.async_copy(rows_v.at[slot], out_hbm.at[b, pl.ds(r0, RC)],
                         osems[slot])

    start_idx_load(0, 0)
    do_chunk(0, 0, True, False)
    do_chunk(1, 1, True, False)

    def pair(h, carry):
        g0 = h * 2
        do_chunk(g0, 0, False, False)
        do_chunk(g0 + 1, 1, False, False)
        return carry

    lax.fori_loop(1, NCHUNK // 2, pair, 0)
    wait_store(0)
    wait_store(1)


from jax.experimental.layout import Format, Layout, with_layout_constraint


@jax.jit
def _run(idx, table):
    mesh = plsc.VectorSubcoreMesh(core_axis_name="c", subcore_axis_name="s")
    f = functools.partial(
        pl.kernel,
        mesh=mesh,
        out_type=jax.ShapeDtypeStruct((B, R, CC, D), jnp.float32),
        scratch_types=[
            pltpu.VMEM((2, RC, CC), jnp.int32),
            pltpu.VMEM((2, RC, CC, D), jnp.float32),
            pltpu.SemaphoreType.DMA,
            pltpu.SemaphoreType.DMA,
            pltpu.SemaphoreType.DMA,
            pltpu.SemaphoreType.DMA,
            pltpu.SemaphoreType.DMA,
        ],
        compiler_params=pltpu.CompilerParams(use_tc_tiling_on_sc=False),
    )(_emb_body)
    return f(idx, table)


def kernel(input, weight):
    out = _run(input, weight)
    lay = Layout(major_to_minor=(0, 1, 2, 3), tiling=((8,), (1024,)))
    return with_layout_constraint(out, lay)
